# baseline (speedup 1.0000x reference)
import jax
import jax.numpy as jnp
from jax import lax
from jax.experimental import pallas as pl
from jax.experimental.pallas import tpu as pltpu
from jax.experimental.pallas import tpu_sc as plsc

_K = 4
_NC = 2
_NS = 16
_L = 16


def _sc_degree(dst_i32, n):
    e = dst_i32.shape[0]
    ept = e // (_NC * _NS)
    rps = n // _NS
    mesh = plsc.VectorSubcoreMesh(core_axis_name="c", subcore_axis_name="s")

    def body(dst_hbm, out_hbm, idx_v, ones_v, zero_v, acc_sh, sem):
        c = lax.axis_index("c")
        s = lax.axis_index("s")
        base = (c * _NS + s) * ept

        @pl.loop(0, rps, step=_L)
        def _(i):
            zero_v[pl.ds(i, _L)] = jnp.zeros((_L,), jnp.float32)

        @pl.loop(0, ept, step=_L)
        def _(i):
            ones_v[pl.ds(i, _L)] = jnp.ones((_L,), jnp.float32)

        pltpu.sync_copy(zero_v, acc_sh.at[pl.ds(s * rps, rps)])
        pltpu.async_copy(dst_hbm.at[pl.ds(base, ept)], idx_v, sem).wait()
        plsc.subcore_barrier()
        pltpu.sync_copy(ones_v, acc_sh.at[idx_v], add=True)
        plsc.subcore_barrier()
        pltpu.sync_copy(acc_sh.at[pl.ds(s * rps, rps)],
                        out_hbm.at[c, pl.ds(s * rps, rps)])

    return pl.kernel(
        body,
        out_type=jax.ShapeDtypeStruct((_NC, n), jnp.float32),
        mesh=mesh,
        scratch_types=[
            pltpu.VMEM((ept,), jnp.int32),
            pltpu.VMEM((ept,), jnp.float32),
            pltpu.VMEM((rps,), jnp.float32),
            pltpu.VMEM_SHARED((n,), jnp.float32),
            pltpu.SemaphoreType.DMA,
        ],
    )(dst_i32)


def _sc_scatter_rows(src_i32, dst_i32, hn0, hn1, n, dh):
    e = src_i32.shape[0]
    chunk = 128
    ept = e // (_NC * _NS)
    npt = ept // chunk
    rps = n // _NS
    src2 = src_i32.reshape(e // chunk, chunk)
    dst2 = dst_i32.reshape(e // chunk, chunk)
    mesh = plsc.VectorSubcoreMesh(core_axis_name="c", subcore_axis_name="s")

    def body(src_hbm, dst_hbm, hn0_hbm, hn1_hbm, out_hbm, sidx, didx, rows0,
             rows1, acc_sh, sem0, sem1):
        c = lax.axis_index("c")
        s = lax.axis_index("s")
        cbase = (c * _NS + s) * npt
        pltpu.sync_copy(src_hbm.at[pl.ds(cbase, npt)], sidx)
        pltpu.sync_copy(dst_hbm.at[pl.ds(cbase, npt)], didx)
        bufs = (rows0, rows1)
        sems = (sem0, sem1)

        for half, hbm in enumerate((hn0_hbm, hn1_hbm)):
            @pl.loop(0, chunk)
            def _(r):
                @pl.loop(0, dh, step=_L)
                def _(j):
                    rows0[r, pl.ds(j, _L)] = jnp.zeros((_L,), jnp.float32)

            @pl.loop(0, rps, step=chunk)
            def _(r0):
                pltpu.sync_copy(rows0, acc_sh.at[pl.ds(s * rps + r0, chunk)])

            plsc.subcore_barrier()

            cps = [None] * npt
            cps[0] = pltpu.async_copy(hbm.at[sidx.at[0]], bufs[0], sems[0])
            for i in range(npt):
                if i + 1 < npt:
                    cps[i + 1] = pltpu.async_copy(
                        hbm.at[sidx.at[i + 1]], bufs[(i + 1) % 2],
                        sems[(i + 1) % 2])
                cps[i].wait()
                pltpu.sync_copy(bufs[i % 2], acc_sh.at[didx.at[i]], add=True)

            plsc.subcore_barrier()
            pltpu.sync_copy(acc_sh.at[pl.ds(s * rps, rps)],
                            out_hbm.at[half, c, pl.ds(s * rps, rps)])

    return pl.kernel(
        body,
        out_type=jax.ShapeDtypeStruct((2, _NC, n, dh), jnp.float32),
        mesh=mesh,
        scratch_types=[
            pltpu.VMEM((npt, chunk), jnp.int32),
            pltpu.VMEM((npt, chunk), jnp.int32),
            pltpu.VMEM((chunk, dh), jnp.float32),
            pltpu.VMEM((chunk, dh), jnp.float32),
            pltpu.VMEM_SHARED((n, dh), jnp.float32),
            pltpu.SemaphoreType.DMA,
            pltpu.SemaphoreType.DMA,
        ],
    )(src2, dst2, hn0, hn1)


def _hn_body(x_ref, w_ref, deg_ref, hn_ref, dinv_ref):
    deg = deg_ref[:, 0:1] + deg_ref[:, 1:2] + 1.0
    dinv = lax.rsqrt(deg)
    h = jnp.dot(x_ref[...], w_ref[...], preferred_element_type=jnp.float32)
    hn_ref[...] = h * dinv
    dinv_ref[...] = dinv


def _tc_hn(x, W, deg2, n, dout):
    return pl.pallas_call(
        _hn_body,
        out_shape=(jax.ShapeDtypeStruct((n, dout), jnp.float32),
                   jax.ShapeDtypeStruct((n, 1), jnp.float32)),
    )(x, W, deg2)


def _combine_body(acc_ref, hn_ref, dinv_ref, b_ref, xe_ref, sq_ref):
    s = jnp.concatenate([acc_ref[0, 0] + acc_ref[0, 1],
                         acc_ref[1, 0] + acc_ref[1, 1]], axis=1)
    xe = dinv_ref[...] * (s + hn_ref[...]) + b_ref[...]
    xe_ref[...] = xe
    sq_ref[...] = jnp.sum(xe * xe, axis=1, keepdims=True)


def _tc_combine(accp, hn, dinv, b2, n, dout):
    return pl.pallas_call(
        _combine_body,
        out_shape=(jax.ShapeDtypeStruct((n, dout), jnp.float32),
                   jax.ShapeDtypeStruct((n, 1), jnp.float32)),
    )(accp, hn, dinv, b2)


_RBLK = 256


def _dist_topk_body(tneg_ref, xe_ref, sqc_ref, sqr_ref, g_ref, tv_ref, ti_ref):
    i = pl.program_id(0)
    n = xe_ref.shape[0]
    xb = xe_ref[pl.ds(i * _RBLK, _RBLK), :]
    sqb = sqc_ref[pl.ds(i * _RBLK, _RBLK), :]
    dot = lax.dot_general(xb, xe_ref[...], (((1,), (1,)), ((), ())),
                          preferred_element_type=jnp.float32)
    d2 = jnp.maximum(sqb + sqr_ref[...] - 2.0 * dot, 0.0)
    work = tneg_ref[...] * d2 + g_ref[...]
    cols = lax.broadcasted_iota(jnp.int32, (_RBLK, n), 1)
    for k in range(_K):
        m = jnp.max(work, axis=1, keepdims=True)
        idx = jnp.min(jnp.where(work == m, cols, n), axis=1, keepdims=True)
        tv_ref[:, k:k + 1] = m
        ti_ref[:, k:k + 1] = idx
        if k + 1 < _K:
            work = jnp.where(cols == idx, -jnp.inf, work)


def _tc_dist_topk(tneg, xe, sqc, sqr, g, n):
    grid = (n // _RBLK,)
    return pl.pallas_call(
        _dist_topk_body,
        grid=grid,
        in_specs=[
            pl.BlockSpec((1, 1), lambda i: (0, 0)),
            pl.BlockSpec((n, xe.shape[1]), lambda i: (0, 0)),
            pl.BlockSpec((n, 1), lambda i: (0, 0)),
            pl.BlockSpec((1, n), lambda i: (0, 0)),
            pl.BlockSpec((_RBLK, n), lambda i: (i, 0)),
        ],
        out_specs=[
            pl.BlockSpec((_RBLK, _K), lambda i: (i, 0)),
            pl.BlockSpec((_RBLK, _K), lambda i: (i, 0)),
        ],
        out_shape=(jax.ShapeDtypeStruct((n, _K), jnp.float32),
                   jax.ShapeDtypeStruct((n, _K), jnp.int32)),
    )(tneg, xe, sqc, sqr, g)


def kernel(x, edge_index, W, b, temperature):
    n, _ = x.shape
    dout = W.shape[1]
    ei = edge_index.astype(jnp.int32)
    src, dst = ei[0], ei[1]

    degp = _sc_degree(dst, n)
    deg2 = degp.T
    hn, dinv = _tc_hn(x, W, deg2, n, dout)
    dh = dout // 2
    accp = _sc_scatter_rows(src, dst, hn[:, :dh], hn[:, dh:], n, dh)
    b2 = b.reshape(1, dout)
    xe, sqc = _tc_combine(accp, hn, dinv, b2, n, dout)
    sqr = sqc.reshape(1, n)

    zero = lax.shift_right_logical(ei[0, 0], 31).astype(jnp.int32)
    q = jax.random.uniform(jax.random.key(jnp.int32(42) + zero), (n, n),
                           dtype=jnp.float32) + 1e-8
    g = -jnp.log(-jnp.log(q))

    tneg = (-temperature).reshape(1, 1)
    topvals, topidx = _tc_dist_topk(tneg, xe, sqc, sqr, g, n)

    ar = jnp.arange(n, dtype=jnp.int32)
    rows = jnp.repeat(ar, _K)
    edges = jnp.stack([topidx.reshape(-1), rows])
    edge_index_hat = jnp.concatenate([edges, jnp.stack([ar, ar])], axis=1)
    return (xe, edge_index_hat, topvals)


# speedup vs baseline: 1.0001x; 1.0001x over previous
import jax
import jax.numpy as jnp
from jax import lax
from jax.experimental import pallas as pl
from jax.experimental.pallas import tpu as pltpu
from jax.experimental.pallas import tpu_sc as plsc

_K = 4
_NC = 2
_NS = 16
_L = 16


def _sc_degree(dst_i32, n):
    e = dst_i32.shape[0]
    ept = e // (_NC * _NS)
    rps = n // _NS
    mesh = plsc.VectorSubcoreMesh(core_axis_name="c", subcore_axis_name="s")

    def body(dst_hbm, out_hbm, idx_v, ones_v, zero_v, acc_sh, sem):
        c = lax.axis_index("c")
        s = lax.axis_index("s")
        base = (c * _NS + s) * ept

        @pl.loop(0, rps, step=_L)
        def _(i):
            zero_v[pl.ds(i, _L)] = jnp.zeros((_L,), jnp.float32)

        @pl.loop(0, ept, step=_L)
        def _(i):
            ones_v[pl.ds(i, _L)] = jnp.ones((_L,), jnp.float32)

        pltpu.sync_copy(zero_v, acc_sh.at[pl.ds(s * rps, rps)])
        pltpu.async_copy(dst_hbm.at[pl.ds(base, ept)], idx_v, sem).wait()
        plsc.subcore_barrier()
        pltpu.sync_copy(ones_v, acc_sh.at[idx_v], add=True)
        plsc.subcore_barrier()
        pltpu.sync_copy(acc_sh.at[pl.ds(s * rps, rps)],
                        out_hbm.at[c, pl.ds(s * rps, rps)])

    return pl.kernel(
        body,
        out_type=jax.ShapeDtypeStruct((_NC, n), jnp.float32),
        mesh=mesh,
        scratch_types=[
            pltpu.VMEM((ept,), jnp.int32),
            pltpu.VMEM((ept,), jnp.float32),
            pltpu.VMEM((rps,), jnp.float32),
            pltpu.VMEM_SHARED((n,), jnp.float32),
            pltpu.SemaphoreType.DMA,
        ],
    )(dst_i32)


def _sc_scatter_rows(src_i32, dst_i32, hn0, hn1, n, dh):
    e = src_i32.shape[0]
    chunk = 128
    ept = e // (_NC * _NS)
    npt = ept // chunk
    rps = n // _NS
    src2 = src_i32.reshape(e // chunk, chunk)
    dst2 = dst_i32.reshape(e // chunk, chunk)
    mesh = plsc.VectorSubcoreMesh(core_axis_name="c", subcore_axis_name="s")

    def body(src_hbm, dst_hbm, hn0_hbm, hn1_hbm, out_hbm, sidx, didx, rows0,
             rows1, acc_sh, sem0, sem1):
        c = lax.axis_index("c")
        s = lax.axis_index("s")
        cbase = (c * _NS + s) * npt
        pltpu.sync_copy(src_hbm.at[pl.ds(cbase, npt)], sidx)
        pltpu.sync_copy(dst_hbm.at[pl.ds(cbase, npt)], didx)
        bufs = (rows0, rows1)
        sems = (sem0, sem1)

        for half, hbm in enumerate((hn0_hbm, hn1_hbm)):
            @pl.loop(0, chunk)
            def _(r):
                @pl.loop(0, dh, step=_L)
                def _(j):
                    rows0[r, pl.ds(j, _L)] = jnp.zeros((_L,), jnp.float32)

            @pl.loop(0, rps, step=chunk)
            def _(r0):
                pltpu.sync_copy(rows0, acc_sh.at[pl.ds(s * rps + r0, chunk)])

            plsc.subcore_barrier()

            cps = [None] * npt
            cps[0] = pltpu.async_copy(hbm.at[sidx.at[0]], bufs[0], sems[0])
            for i in range(npt):
                if i + 1 < npt:
                    cps[i + 1] = pltpu.async_copy(
                        hbm.at[sidx.at[i + 1]], bufs[(i + 1) % 2],
                        sems[(i + 1) % 2])
                cps[i].wait()
                pltpu.sync_copy(bufs[i % 2], acc_sh.at[didx.at[i]], add=True)

            plsc.subcore_barrier()
            pltpu.sync_copy(acc_sh.at[pl.ds(s * rps, rps)],
                            out_hbm.at[half, c, pl.ds(s * rps, rps)])

    return pl.kernel(
        body,
        out_type=jax.ShapeDtypeStruct((2, _NC, n, dh), jnp.float32),
        mesh=mesh,
        scratch_types=[
            pltpu.VMEM((npt, chunk), jnp.int32),
            pltpu.VMEM((npt, chunk), jnp.int32),
            pltpu.VMEM((chunk, dh), jnp.float32),
            pltpu.VMEM((chunk, dh), jnp.float32),
            pltpu.VMEM_SHARED((n, dh), jnp.float32),
            pltpu.SemaphoreType.DMA,
            pltpu.SemaphoreType.DMA,
        ],
    )(src2, dst2, hn0, hn1)


def _hn_body(x_ref, w_ref, deg_ref, hn_ref, dinv_ref):
    deg = deg_ref[:, 0:1] + deg_ref[:, 1:2] + 1.0
    dinv = lax.rsqrt(deg)
    h = jnp.dot(x_ref[...], w_ref[...], preferred_element_type=jnp.float32)
    hn_ref[...] = h * dinv
    dinv_ref[...] = dinv


def _tc_hn(x, W, deg2, n, dout):
    return pl.pallas_call(
        _hn_body,
        out_shape=(jax.ShapeDtypeStruct((n, dout), jnp.float32),
                   jax.ShapeDtypeStruct((n, 1), jnp.float32)),
    )(x, W, deg2)


def _combine_body(acc_ref, hn_ref, dinv_ref, b_ref, xe_ref, sq_ref):
    s = jnp.concatenate([acc_ref[0, 0] + acc_ref[0, 1],
                         acc_ref[1, 0] + acc_ref[1, 1]], axis=1)
    xe = dinv_ref[...] * (s + hn_ref[...]) + b_ref[...]
    xe_ref[...] = xe
    sq_ref[...] = jnp.sum(xe * xe, axis=1, keepdims=True)


def _tc_combine(accp, hn, dinv, b2, n, dout):
    return pl.pallas_call(
        _combine_body,
        out_shape=(jax.ShapeDtypeStruct((n, dout), jnp.float32),
                   jax.ShapeDtypeStruct((n, 1), jnp.float32)),
    )(accp, hn, dinv, b2)


_RBLK = 256


def _dist_topk_body(tneg_ref, xe_ref, sqc_ref, sqr_ref, g_ref, tv_ref, ti_ref):
    i = pl.program_id(0)
    n = xe_ref.shape[0]
    xb = xe_ref[pl.ds(i * _RBLK, _RBLK), :]
    sqb = sqc_ref[pl.ds(i * _RBLK, _RBLK), :]
    dot = lax.dot_general(xb, xe_ref[...], (((1,), (1,)), ((), ())),
                          preferred_element_type=jnp.float32)
    d2 = jnp.maximum(sqb + sqr_ref[...] - 2.0 * dot, 0.0)
    work = tneg_ref[...] * d2 + g_ref[...]
    cols = lax.broadcasted_iota(jnp.int32, (_RBLK, n), 1)
    for k in range(_K):
        m = jnp.max(work, axis=1, keepdims=True)
        idx = jnp.min(jnp.where(work == m, cols, n), axis=1, keepdims=True)
        tv_ref[:, k:k + 1] = m
        ti_ref[:, k:k + 1] = idx
        if k + 1 < _K:
            work = jnp.where(cols == idx, -jnp.inf, work)


def _tc_dist_topk(tneg, xe, sqc, sqr, g, n):
    grid = (n // _RBLK,)
    return pl.pallas_call(
        _dist_topk_body,
        grid=grid,
        in_specs=[
            pl.BlockSpec((1, 1), lambda i: (0, 0)),
            pl.BlockSpec((n, xe.shape[1]), lambda i: (0, 0)),
            pl.BlockSpec((n, 1), lambda i: (0, 0)),
            pl.BlockSpec((1, n), lambda i: (0, 0)),
            pl.BlockSpec((_RBLK, n), lambda i: (i, 0)),
        ],
        out_specs=[
            pl.BlockSpec((_RBLK, _K), lambda i: (i, 0)),
            pl.BlockSpec((_RBLK, _K), lambda i: (i, 0)),
        ],
        out_shape=(jax.ShapeDtypeStruct((n, _K), jnp.float32),
                   jax.ShapeDtypeStruct((n, _K), jnp.int32)),
    )(tneg, xe, sqc, sqr, g)


def kernel(x, edge_index, W, b, temperature):
    n, _ = x.shape
    dout = W.shape[1]
    ei = edge_index.astype(jnp.int32)
    src, dst = ei[0], ei[1]

    degp = _sc_degree(dst, n)
    deg2 = degp.T
    hn, dinv = _tc_hn(x, W, deg2, n, dout)
    dh = dout // 2
    accp = _sc_scatter_rows(src, dst, hn[:, :dh], hn[:, dh:], n, dh)

    zero = lax.shift_right_logical(ei[0, 0], 31).astype(jnp.int32)
    q = jax.random.uniform(jax.random.key(jnp.int32(42) + zero), (n, n),
                           dtype=jnp.float32) + 1e-8
    g = -jnp.log(-jnp.log(q))

    b2 = b.reshape(1, dout)
    xe, sqc = _tc_combine(accp, hn, dinv, b2, n, dout)
    sqr = sqc.reshape(1, n)

    tneg = (-temperature).reshape(1, 1)
    topvals, topidx = _tc_dist_topk(tneg, xe, sqc, sqr, g, n)

    ar = jnp.arange(n, dtype=jnp.int32)
    rows = jnp.repeat(ar, _K)
    edges = jnp.stack([topidx.reshape(-1), rows])
    edge_index_hat = jnp.concatenate([edges, jnp.stack([ar, ar])], axis=1)
    return (xe, edge_index_hat, topvals)


# speedup vs baseline: 1.0398x; 1.0396x over previous
import jax
import jax.numpy as jnp
from jax import lax
from jax.experimental import pallas as pl
from jax.experimental.pallas import tpu as pltpu
from jax.experimental.pallas import tpu_sc as plsc

_K = 4
_NC = 2
_NS = 16
_L = 16


def _sc_degree(dst_i32, n):
    e = dst_i32.shape[0]
    ept = e // (_NC * _NS)
    rps = n // _NS
    mesh = plsc.VectorSubcoreMesh(core_axis_name="c", subcore_axis_name="s")

    def body(dst_hbm, out_hbm, idx_v, ones_v, zero_v, acc_sh, sem):
        c = lax.axis_index("c")
        s = lax.axis_index("s")
        base = (c * _NS + s) * ept

        @pl.loop(0, rps, step=_L)
        def _(i):
            zero_v[pl.ds(i, _L)] = jnp.zeros((_L,), jnp.float32)

        @pl.loop(0, ept, step=_L)
        def _(i):
            ones_v[pl.ds(i, _L)] = jnp.ones((_L,), jnp.float32)

        pltpu.sync_copy(zero_v, acc_sh.at[pl.ds(s * rps, rps)])
        pltpu.async_copy(dst_hbm.at[pl.ds(base, ept)], idx_v, sem).wait()
        plsc.subcore_barrier()
        pltpu.sync_copy(ones_v, acc_sh.at[idx_v], add=True)
        plsc.subcore_barrier()
        pltpu.sync_copy(acc_sh.at[pl.ds(s * rps, rps)],
                        out_hbm.at[c, pl.ds(s * rps, rps)])

    return pl.kernel(
        body,
        out_type=jax.ShapeDtypeStruct((_NC, n), jnp.float32),
        mesh=mesh,
        scratch_types=[
            pltpu.VMEM((ept,), jnp.int32),
            pltpu.VMEM((ept,), jnp.float32),
            pltpu.VMEM((rps,), jnp.float32),
            pltpu.VMEM_SHARED((n,), jnp.float32),
            pltpu.SemaphoreType.DMA,
        ],
    )(dst_i32)


def _sc_scatter_rows(src_i32, dst_i32, hn0, hn1, n, dh):
    e = src_i32.shape[0]
    chunk = 128
    ept = e // (_NC * _NS)
    npt = ept // chunk
    rps = n // _NS
    src2 = src_i32.reshape(e // chunk, chunk)
    dst2 = dst_i32.reshape(e // chunk, chunk)
    mesh = plsc.VectorSubcoreMesh(core_axis_name="c", subcore_axis_name="s")

    def body(src_hbm, dst_hbm, hn0_hbm, hn1_hbm, out_hbm, sidx, didx, rows0,
             rows1, acc_sh, sem0, sem1):
        c = lax.axis_index("c")
        s = lax.axis_index("s")
        cbase = (c * _NS + s) * npt
        pltpu.sync_copy(src_hbm.at[pl.ds(cbase, npt)], sidx)
        pltpu.sync_copy(dst_hbm.at[pl.ds(cbase, npt)], didx)
        bufs = (rows0, rows1)
        sems = (sem0, sem1)

        for half, hbm in enumerate((hn0_hbm, hn1_hbm)):
            @pl.loop(0, chunk)
            def _(r):
                @pl.loop(0, dh, step=_L)
                def _(j):
                    rows0[r, pl.ds(j, _L)] = jnp.zeros((_L,), jnp.float32)

            @pl.loop(0, rps, step=chunk)
            def _(r0):
                pltpu.sync_copy(rows0, acc_sh.at[pl.ds(s * rps + r0, chunk)])

            plsc.subcore_barrier()

            cps = [None] * npt
            cps[0] = pltpu.async_copy(hbm.at[sidx.at[0]], bufs[0], sems[0])
            for i in range(npt):
                if i + 1 < npt:
                    cps[i + 1] = pltpu.async_copy(
                        hbm.at[sidx.at[i + 1]], bufs[(i + 1) % 2],
                        sems[(i + 1) % 2])
                cps[i].wait()
                pltpu.sync_copy(bufs[i % 2], acc_sh.at[didx.at[i]], add=True)

            plsc.subcore_barrier()
            pltpu.sync_copy(acc_sh.at[pl.ds(s * rps, rps)],
                            out_hbm.at[half, c, pl.ds(s * rps, rps)])

    return pl.kernel(
        body,
        out_type=jax.ShapeDtypeStruct((2, _NC, n, dh), jnp.float32),
        mesh=mesh,
        scratch_types=[
            pltpu.VMEM((npt, chunk), jnp.int32),
            pltpu.VMEM((npt, chunk), jnp.int32),
            pltpu.VMEM((chunk, dh), jnp.float32),
            pltpu.VMEM((chunk, dh), jnp.float32),
            pltpu.VMEM_SHARED((n, dh), jnp.float32),
            pltpu.SemaphoreType.DMA,
            pltpu.SemaphoreType.DMA,
        ],
    )(src2, dst2, hn0, hn1)


def _hn_body(x_ref, w_ref, deg_ref, hn_ref, dinv_ref):
    deg = deg_ref[:, 0:1] + deg_ref[:, 1:2] + 1.0
    dinv = lax.rsqrt(deg)
    h = jnp.dot(x_ref[...], w_ref[...], preferred_element_type=jnp.float32)
    hn_ref[...] = h * dinv
    dinv_ref[...] = dinv


def _tc_hn(x, W, deg2, n, dout):
    return pl.pallas_call(
        _hn_body,
        out_shape=(jax.ShapeDtypeStruct((n, dout), jnp.float32),
                   jax.ShapeDtypeStruct((n, 1), jnp.float32)),
    )(x, W, deg2)


def _combine_body(acc_ref, hn_ref, dinv_ref, b_ref, xe_ref, sq_ref):
    s = jnp.concatenate([acc_ref[0, 0] + acc_ref[0, 1],
                         acc_ref[1, 0] + acc_ref[1, 1]], axis=1)
    xe = dinv_ref[...] * (s + hn_ref[...]) + b_ref[...]
    xe_ref[...] = xe
    sq_ref[...] = jnp.sum(xe * xe, axis=1, keepdims=True)


def _tc_combine(accp, hn, dinv, b2, n, dout):
    return pl.pallas_call(
        _combine_body,
        out_shape=(jax.ShapeDtypeStruct((n, dout), jnp.float32),
                   jax.ShapeDtypeStruct((n, 1), jnp.float32)),
    )(accp, hn, dinv, b2)


_RBLK = 256


def _dist_topk_body(tneg_ref, xe_ref, sqc_ref, sqr_ref, g_ref, tv_ref, ti_ref):
    i = pl.program_id(0)
    n = xe_ref.shape[0]
    xb = xe_ref[pl.ds(i * _RBLK, _RBLK), :]
    sqb = sqc_ref[pl.ds(i * _RBLK, _RBLK), :]
    dot = lax.dot_general(xb, xe_ref[...], (((1,), (1,)), ((), ())),
                          preferred_element_type=jnp.float32)
    d2 = jnp.maximum(sqb + sqr_ref[...] - 2.0 * dot, 0.0)
    work = tneg_ref[...] * d2 + g_ref[...]
    cols = lax.broadcasted_iota(jnp.int32, (_RBLK, n), 1)
    for k in range(_K):
        m = jnp.max(work, axis=1, keepdims=True)
        idx = jnp.min(jnp.where(work == m, cols, n), axis=1, keepdims=True)
        tv_ref[:, k:k + 1] = m
        ti_ref[:, k:k + 1] = idx
        if k + 1 < _K:
            work = jnp.where(cols == idx, -jnp.inf, work)


def _tc_dist_topk(tneg, xe, sqc, sqr, g, n):
    grid = (n // _RBLK,)
    return pl.pallas_call(
        _dist_topk_body,
        grid=grid,
        in_specs=[
            pl.BlockSpec((1, 1), lambda i: (0, 0)),
            pl.BlockSpec((n, xe.shape[1]), lambda i: (0, 0)),
            pl.BlockSpec((n, 1), lambda i: (0, 0)),
            pl.BlockSpec((1, n), lambda i: (0, 0)),
            pl.BlockSpec((_RBLK, n), lambda i: (i, 0)),
        ],
        out_specs=[
            pl.BlockSpec((_RBLK, _K), lambda i: (i, 0)),
            pl.BlockSpec((_RBLK, _K), lambda i: (i, 0)),
        ],
        out_shape=(jax.ShapeDtypeStruct((n, _K), jnp.float32),
                   jax.ShapeDtypeStruct((n, _K), jnp.int32)),
    )(tneg, xe, sqc, sqr, g)


def kernel(x, edge_index, W, b, temperature):
    n, _ = x.shape
    dout = W.shape[1]
    ei = edge_index.astype(jnp.int32)
    src, dst = ei[0], ei[1]

    degp = _sc_degree(dst, n)
    deg2 = degp.T
    hn, dinv = _tc_hn(x, W, deg2, n, dout)
    dh = dout // 2
    accp = _sc_scatter_rows(src, dst, hn[:, :dh], hn[:, dh:], n, dh)

    q = jax.random.uniform(jax.random.key(42), (n, n), dtype=jnp.float32) + 1e-8
    g = -jnp.log(-jnp.log(q))

    b2 = b.reshape(1, dout)
    xe, sqc = _tc_combine(accp, hn, dinv, b2, n, dout)
    sqr = sqc.reshape(1, n)

    tneg = (-temperature).reshape(1, 1)
    topvals, topidx = _tc_dist_topk(tneg, xe, sqc, sqr, g, n)

    ar = jnp.arange(n, dtype=jnp.int32)
    rows = jnp.repeat(ar, _K)
    edges = jnp.stack([topidx.reshape(-1), rows])
    edge_index_hat = jnp.concatenate([edges, jnp.stack([ar, ar])], axis=1)
    return (xe, edge_index_hat, topvals)


# speedup vs baseline: 2.2196x; 2.1347x over previous
import numpy as np

import jax
import jax.numpy as jnp
from jax import lax
from jax.experimental import pallas as pl
from jax.experimental.pallas import tpu as pltpu
from jax.experimental.pallas import tpu_sc as plsc

_K = 4
_NC = 2
_NS = 16
_L = 16


def _np_threefry2x32(k1, k2, x0, x1):
    rot = [13, 15, 26, 6, 17, 29, 16, 24]
    ks = [np.uint32(k1), np.uint32(k2),
          np.uint32(k1) ^ np.uint32(k2) ^ np.uint32(0x1BD11BDA)]
    x0 = x0 + ks[0]
    x1 = x1 + ks[1]

    def rotl(x, d):
        return (x << np.uint32(d)) | (x >> np.uint32(32 - d))

    for i in range(5):
        for j in range(4):
            r = rot[(i % 2) * 4 + j]
            x0 = x0 + x1
            x1 = rotl(x1, r)
            x1 = x1 ^ x0
        x0 = x0 + ks[(i + 1) % 3]
        x1 = x1 + ks[(i + 2) % 3] + np.uint32(i + 1)
    return x0, x1


_GUMBEL_CACHE = {}


def _gumbel_table(n):
    if n not in _GUMBEL_CACHE:
        c = np.arange(n * n, dtype=np.uint64)
        o0, o1 = _np_threefry2x32(
            np.uint32(0), np.uint32(42),
            (c >> np.uint64(32)).astype(np.uint32), c.astype(np.uint32))
        bits = o0 ^ o1
        f = ((bits >> np.uint32(9)) | np.uint32(0x3F800000)).view(np.float32)
        u = np.maximum(np.float32(0.0), f - np.float32(1.0))
        q = u + np.float32(1e-8)
        _GUMBEL_CACHE[n] = (-np.log(-np.log(q))).reshape(n, n)
    return _GUMBEL_CACHE[n]


def _sc_degree(dst_i32, n):
    e = dst_i32.shape[0]
    ept = e // (_NC * _NS)
    rps = n // _NS
    mesh = plsc.VectorSubcoreMesh(core_axis_name="c", subcore_axis_name="s")

    def body(dst_hbm, out_hbm, idx_v, ones_v, zero_v, acc_sh, sem):
        c = lax.axis_index("c")
        s = lax.axis_index("s")
        base = (c * _NS + s) * ept

        @pl.loop(0, rps, step=_L)
        def _(i):
            zero_v[pl.ds(i, _L)] = jnp.zeros((_L,), jnp.float32)

        @pl.loop(0, ept, step=_L)
        def _(i):
            ones_v[pl.ds(i, _L)] = jnp.ones((_L,), jnp.float32)

        pltpu.sync_copy(zero_v, acc_sh.at[pl.ds(s * rps, rps)])
        pltpu.async_copy(dst_hbm.at[pl.ds(base, ept)], idx_v, sem).wait()
        plsc.subcore_barrier()
        pltpu.sync_copy(ones_v, acc_sh.at[idx_v], add=True)
        plsc.subcore_barrier()
        pltpu.sync_copy(acc_sh.at[pl.ds(s * rps, rps)],
                        out_hbm.at[c, pl.ds(s * rps, rps)])

    return pl.kernel(
        body,
        out_type=jax.ShapeDtypeStruct((_NC, n), jnp.float32),
        mesh=mesh,
        scratch_types=[
            pltpu.VMEM((ept,), jnp.int32),
            pltpu.VMEM((ept,), jnp.float32),
            pltpu.VMEM((rps,), jnp.float32),
            pltpu.VMEM_SHARED((n,), jnp.float32),
            pltpu.SemaphoreType.DMA,
        ],
    )(dst_i32)


def _sc_scatter_rows(src_i32, dst_i32, hn0, hn1, n, dh):
    e = src_i32.shape[0]
    chunk = 128
    ept = e // (_NC * _NS)
    npt = ept // chunk
    rps = n // _NS
    src2 = src_i32.reshape(e // chunk, chunk)
    dst2 = dst_i32.reshape(e // chunk, chunk)
    mesh = plsc.VectorSubcoreMesh(core_axis_name="c", subcore_axis_name="s")

    def body(src_hbm, dst_hbm, hn0_hbm, hn1_hbm, out_hbm, sidx, didx, rows0,
             rows1, acc_sh, sem0, sem1):
        c = lax.axis_index("c")
        s = lax.axis_index("s")
        cbase = (c * _NS + s) * npt
        pltpu.sync_copy(src_hbm.at[pl.ds(cbase, npt)], sidx)
        pltpu.sync_copy(dst_hbm.at[pl.ds(cbase, npt)], didx)
        bufs = (rows0, rows1)
        sems = (sem0, sem1)

        for half, hbm in enumerate((hn0_hbm, hn1_hbm)):
            @pl.loop(0, chunk)
            def _(r):
                @pl.loop(0, dh, step=_L)
                def _(j):
                    rows0[r, pl.ds(j, _L)] = jnp.zeros((_L,), jnp.float32)

            @pl.loop(0, rps, step=chunk)
            def _(r0):
                pltpu.sync_copy(rows0, acc_sh.at[pl.ds(s * rps + r0, chunk)])

            plsc.subcore_barrier()

            cps = [None] * npt
            cps[0] = pltpu.async_copy(hbm.at[sidx.at[0]], bufs[0], sems[0])
            for i in range(npt):
                if i + 1 < npt:
                    cps[i + 1] = pltpu.async_copy(
                        hbm.at[sidx.at[i + 1]], bufs[(i + 1) % 2],
                        sems[(i + 1) % 2])
                cps[i].wait()
                pltpu.sync_copy(bufs[i % 2], acc_sh.at[didx.at[i]], add=True)

            plsc.subcore_barrier()
            pltpu.sync_copy(acc_sh.at[pl.ds(s * rps, rps)],
                            out_hbm.at[half, c, pl.ds(s * rps, rps)])

    return pl.kernel(
        body,
        out_type=jax.ShapeDtypeStruct((2, _NC, n, dh), jnp.float32),
        mesh=mesh,
        scratch_types=[
            pltpu.VMEM((npt, chunk), jnp.int32),
            pltpu.VMEM((npt, chunk), jnp.int32),
            pltpu.VMEM((chunk, dh), jnp.float32),
            pltpu.VMEM((chunk, dh), jnp.float32),
            pltpu.VMEM_SHARED((n, dh), jnp.float32),
            pltpu.SemaphoreType.DMA,
            pltpu.SemaphoreType.DMA,
        ],
    )(src2, dst2, hn0, hn1)


def _hn_body(x_ref, w_ref, deg_ref, hn_ref, dinv_ref):
    deg = deg_ref[:, 0:1] + deg_ref[:, 1:2] + 1.0
    dinv = lax.rsqrt(deg)
    h = jnp.dot(x_ref[...], w_ref[...], preferred_element_type=jnp.float32)
    hn_ref[...] = h * dinv
    dinv_ref[...] = dinv


def _tc_hn(x, W, deg2, n, dout):
    return pl.pallas_call(
        _hn_body,
        out_shape=(jax.ShapeDtypeStruct((n, dout), jnp.float32),
                   jax.ShapeDtypeStruct((n, 1), jnp.float32)),
    )(x, W, deg2)


def _combine_body(acc_ref, hn_ref, dinv_ref, b_ref, xe_ref, sq_ref):
    s = jnp.concatenate([acc_ref[0, 0] + acc_ref[0, 1],
                         acc_ref[1, 0] + acc_ref[1, 1]], axis=1)
    xe = dinv_ref[...] * (s + hn_ref[...]) + b_ref[...]
    xe_ref[...] = xe
    sq_ref[...] = jnp.sum(xe * xe, axis=1, keepdims=True)


def _tc_combine(accp, hn, dinv, b2, n, dout):
    return pl.pallas_call(
        _combine_body,
        out_shape=(jax.ShapeDtypeStruct((n, dout), jnp.float32),
                   jax.ShapeDtypeStruct((n, 1), jnp.float32)),
    )(accp, hn, dinv, b2)


_RBLK = 256


def _dist_topk_body(tneg_ref, xe_ref, sqc_ref, sqr_ref, g_ref, tv_ref, ti_ref):
    i = pl.program_id(0)
    n = xe_ref.shape[0]
    xb = xe_ref[pl.ds(i * _RBLK, _RBLK), :]
    sqb = sqc_ref[pl.ds(i * _RBLK, _RBLK), :]
    dot = lax.dot_general(xb, xe_ref[...], (((1,), (1,)), ((), ())),
                          preferred_element_type=jnp.float32)
    d2 = jnp.maximum(sqb + sqr_ref[...] - 2.0 * dot, 0.0)
    work = tneg_ref[...] * d2 + g_ref[...]
    cols = lax.broadcasted_iota(jnp.int32, (_RBLK, n), 1)
    for k in range(_K):
        m = jnp.max(work, axis=1, keepdims=True)
        idx = jnp.min(jnp.where(work == m, cols, n), axis=1, keepdims=True)
        tv_ref[:, k:k + 1] = m
        ti_ref[:, k:k + 1] = idx
        if k + 1 < _K:
            work = jnp.where(cols == idx, -jnp.inf, work)


def _tc_dist_topk(tneg, xe, sqc, sqr, g, n):
    grid = (n // _RBLK,)
    return pl.pallas_call(
        _dist_topk_body,
        grid=grid,
        in_specs=[
            pl.BlockSpec((1, 1), lambda i: (0, 0)),
            pl.BlockSpec((n, xe.shape[1]), lambda i: (0, 0)),
            pl.BlockSpec((n, 1), lambda i: (0, 0)),
            pl.BlockSpec((1, n), lambda i: (0, 0)),
            pl.BlockSpec((_RBLK, n), lambda i: (i, 0)),
        ],
        out_specs=[
            pl.BlockSpec((_RBLK, _K), lambda i: (i, 0)),
            pl.BlockSpec((_RBLK, _K), lambda i: (i, 0)),
        ],
        out_shape=(jax.ShapeDtypeStruct((n, _K), jnp.float32),
                   jax.ShapeDtypeStruct((n, _K), jnp.int32)),
    )(tneg, xe, sqc, sqr, g)


def kernel(x, edge_index, W, b, temperature):
    n, _ = x.shape
    dout = W.shape[1]
    ei = edge_index.astype(jnp.int32)
    src, dst = ei[0], ei[1]

    degp = _sc_degree(dst, n)
    deg2 = degp.T
    hn, dinv = _tc_hn(x, W, deg2, n, dout)
    dh = dout // 2
    accp = _sc_scatter_rows(src, dst, hn[:, :dh], hn[:, dh:], n, dh)

    g = jnp.asarray(_gumbel_table(n))

    b2 = b.reshape(1, dout)
    xe, sqc = _tc_combine(accp, hn, dinv, b2, n, dout)
    sqr = sqc.reshape(1, n)

    tneg = (-temperature).reshape(1, 1)
    topvals, topidx = _tc_dist_topk(tneg, xe, sqc, sqr, g, n)

    ar = jnp.arange(n, dtype=jnp.int32)
    rows = jnp.repeat(ar, _K)
    edges = jnp.stack([topidx.reshape(-1), rows])
    edge_index_hat = jnp.concatenate([edges, jnp.stack([ar, ar])], axis=1)
    return (xe, edge_index_hat, topvals)


# speedup vs baseline: 2.3454x; 1.0567x over previous
import numpy as np

import jax
import jax.numpy as jnp
from jax import lax
from jax.experimental import pallas as pl
from jax.experimental.pallas import tpu as pltpu
from jax.experimental.pallas import tpu_sc as plsc

_K = 4
_NC = 2
_NS = 16
_L = 16


def _np_threefry2x32(k1, k2, x0, x1):
    rot = [13, 15, 26, 6, 17, 29, 16, 24]
    ks = [np.uint32(k1), np.uint32(k2),
          np.uint32(k1) ^ np.uint32(k2) ^ np.uint32(0x1BD11BDA)]
    x0 = x0 + ks[0]
    x1 = x1 + ks[1]

    def rotl(x, d):
        return (x << np.uint32(d)) | (x >> np.uint32(32 - d))

    for i in range(5):
        for j in range(4):
            r = rot[(i % 2) * 4 + j]
            x0 = x0 + x1
            x1 = rotl(x1, r)
            x1 = x1 ^ x0
        x0 = x0 + ks[(i + 1) % 3]
        x1 = x1 + ks[(i + 2) % 3] + np.uint32(i + 1)
    return x0, x1


_GUMBEL_CACHE = {}


def _gumbel_table(n):
    if n not in _GUMBEL_CACHE:
        c = np.arange(n * n, dtype=np.uint64)
        o0, o1 = _np_threefry2x32(
            np.uint32(0), np.uint32(42),
            (c >> np.uint64(32)).astype(np.uint32), c.astype(np.uint32))
        bits = o0 ^ o1
        f = ((bits >> np.uint32(9)) | np.uint32(0x3F800000)).view(np.float32)
        u = np.maximum(np.float32(0.0), f - np.float32(1.0))
        q = u + np.float32(1e-8)
        _GUMBEL_CACHE[n] = (-np.log(-np.log(q))).reshape(n, n)
    return _GUMBEL_CACHE[n]


def _sc_degree(dst_i32, n):
    e = dst_i32.shape[0]
    ept = e // (_NC * _NS)
    rps = n // _NS
    mesh = plsc.VectorSubcoreMesh(core_axis_name="c", subcore_axis_name="s")

    def body(dst_hbm, out_hbm, idx_v, ones_v, zero_v, acc_sh, sem):
        c = lax.axis_index("c")
        s = lax.axis_index("s")
        base = (c * _NS + s) * ept

        @pl.loop(0, rps, step=_L)
        def _(i):
            zero_v[pl.ds(i, _L)] = jnp.zeros((_L,), jnp.float32)

        @pl.loop(0, ept, step=_L)
        def _(i):
            ones_v[pl.ds(i, _L)] = jnp.ones((_L,), jnp.float32)

        pltpu.sync_copy(zero_v, acc_sh.at[pl.ds(s * rps, rps)])
        pltpu.async_copy(dst_hbm.at[pl.ds(base, ept)], idx_v, sem).wait()
        plsc.subcore_barrier()
        pltpu.sync_copy(ones_v, acc_sh.at[idx_v], add=True)
        plsc.subcore_barrier()
        pltpu.sync_copy(acc_sh.at[pl.ds(s * rps, rps)],
                        out_hbm.at[c, pl.ds(s * rps, rps)])

    return pl.kernel(
        body,
        out_type=jax.ShapeDtypeStruct((_NC, n), jnp.float32),
        mesh=mesh,
        scratch_types=[
            pltpu.VMEM((ept,), jnp.int32),
            pltpu.VMEM((ept,), jnp.float32),
            pltpu.VMEM((rps,), jnp.float32),
            pltpu.VMEM_SHARED((n,), jnp.float32),
            pltpu.SemaphoreType.DMA,
        ],
    )(dst_i32)


def _sc_scatter_rows(src_i32, dst_i32, hn0, hn1, n, dh):
    e = src_i32.shape[0]
    chunk = 128
    ept = e // (_NC * _NS)
    npt = ept // chunk
    rps = n // _NS
    src2 = src_i32.reshape(e // chunk, chunk)
    dst2 = dst_i32.reshape(e // chunk, chunk)
    mesh = plsc.VectorSubcoreMesh(core_axis_name="c", subcore_axis_name="s")

    nbuf = 4

    def body(src_hbm, dst_hbm, hn0_hbm, hn1_hbm, out_hbm, sidx, didx, rows0,
             rows1, rows2, rows3, acc_sh, sem0, sem1, sem2, sem3):
        c = lax.axis_index("c")
        s = lax.axis_index("s")
        cbase = (c * _NS + s) * npt
        pltpu.sync_copy(src_hbm.at[pl.ds(cbase, npt)], sidx)
        pltpu.sync_copy(dst_hbm.at[pl.ds(cbase, npt)], didx)
        bufs = (rows0, rows1, rows2, rows3)
        sems = (sem0, sem1, sem2, sem3)

        for half, hbm in enumerate((hn0_hbm, hn1_hbm)):
            @pl.loop(0, chunk)
            def _(r):
                @pl.loop(0, dh, step=_L)
                def _(j):
                    rows0[r, pl.ds(j, _L)] = jnp.zeros((_L,), jnp.float32)

            @pl.loop(0, rps, step=chunk)
            def _(r0):
                pltpu.sync_copy(rows0, acc_sh.at[pl.ds(s * rps + r0, chunk)])

            plsc.subcore_barrier()

            cps = [None] * npt
            for i in range(nbuf - 1):
                cps[i] = pltpu.async_copy(hbm.at[sidx.at[i]], bufs[i % nbuf],
                                          sems[i % nbuf])
            for i in range(npt):
                if i + nbuf - 1 < npt:
                    j = i + nbuf - 1
                    cps[j] = pltpu.async_copy(hbm.at[sidx.at[j]],
                                              bufs[j % nbuf], sems[j % nbuf])
                cps[i].wait()
                pltpu.sync_copy(bufs[i % nbuf], acc_sh.at[didx.at[i]],
                                add=True)

            plsc.subcore_barrier()
            pltpu.sync_copy(acc_sh.at[pl.ds(s * rps, rps)],
                            out_hbm.at[half, c, pl.ds(s * rps, rps)])

    return pl.kernel(
        body,
        out_type=jax.ShapeDtypeStruct((2, _NC, n, dh), jnp.float32),
        mesh=mesh,
        scratch_types=[
            pltpu.VMEM((npt, chunk), jnp.int32),
            pltpu.VMEM((npt, chunk), jnp.int32),
            pltpu.VMEM((chunk, dh), jnp.float32),
            pltpu.VMEM((chunk, dh), jnp.float32),
            pltpu.VMEM((chunk, dh), jnp.float32),
            pltpu.VMEM((chunk, dh), jnp.float32),
            pltpu.VMEM_SHARED((n, dh), jnp.float32),
            pltpu.SemaphoreType.DMA,
            pltpu.SemaphoreType.DMA,
            pltpu.SemaphoreType.DMA,
            pltpu.SemaphoreType.DMA,
        ],
    )(src2, dst2, hn0, hn1)


def _hn_body(x_ref, w_ref, degp_ref, ones_ref, hn0_ref, hn1_ref, dinv_ref):
    deg = lax.dot_general(degp_ref[...], ones_ref[...],
                          (((0,), (0,)), ((), ())),
                          preferred_element_type=jnp.float32) + 1.0
    dinv = lax.rsqrt(deg)
    h = jnp.dot(x_ref[...], w_ref[...], preferred_element_type=jnp.float32)
    hn = h * dinv
    dh = hn0_ref.shape[1]
    hn0_ref[...] = hn[:, :dh]
    hn1_ref[...] = hn[:, dh:]
    dinv_ref[...] = dinv


def _tc_hn(x, W, degp, n, dout):
    ones = jnp.ones((2, 1), jnp.float32)
    dh = dout // 2
    return pl.pallas_call(
        _hn_body,
        out_shape=(jax.ShapeDtypeStruct((n, dh), jnp.float32),
                   jax.ShapeDtypeStruct((n, dh), jnp.float32),
                   jax.ShapeDtypeStruct((n, 1), jnp.float32)),
    )(x, W, degp, ones)


def _combine_body(acc_ref, hn0_ref, hn1_ref, dinv_ref, b_ref, xe_ref, sq_ref):
    s = jnp.concatenate([acc_ref[0, 0] + acc_ref[0, 1] + hn0_ref[...],
                         acc_ref[1, 0] + acc_ref[1, 1] + hn1_ref[...]], axis=1)
    xe = dinv_ref[...] * s + b_ref[...]
    xe_ref[...] = xe
    sq_ref[...] = jnp.sum(xe * xe, axis=1, keepdims=True)


def _tc_combine(accp, hn0, hn1, dinv, b2, n, dout):
    return pl.pallas_call(
        _combine_body,
        out_shape=(jax.ShapeDtypeStruct((n, dout), jnp.float32),
                   jax.ShapeDtypeStruct((n, 1), jnp.float32)),
    )(accp, hn0, hn1, dinv, b2)


_RBLK = 256


def _dist_topk_body(tneg_ref, xe_ref, sqc_ref, sqr_ref, g_ref, tv_ref, ti_ref):
    i = pl.program_id(0)
    n = xe_ref.shape[0]
    xb = xe_ref[pl.ds(i * _RBLK, _RBLK), :]
    sqb = sqc_ref[pl.ds(i * _RBLK, _RBLK), :]
    dot = lax.dot_general(xb, xe_ref[...], (((1,), (1,)), ((), ())),
                          preferred_element_type=jnp.float32)
    d2 = jnp.maximum(sqb + sqr_ref[...] - 2.0 * dot, 0.0)
    work = tneg_ref[...] * d2 + g_ref[...]
    cols = lax.broadcasted_iota(jnp.int32, (_RBLK, n), 1)
    for k in range(_K):
        m = jnp.max(work, axis=1, keepdims=True)
        idx = jnp.min(jnp.where(work == m, cols, n), axis=1, keepdims=True)
        tv_ref[:, k:k + 1] = m
        ti_ref[:, k:k + 1] = idx
        if k + 1 < _K:
            work = jnp.where(cols == idx, -jnp.inf, work)


def _tc_dist_topk(tneg, xe, sqc, sqr, g, n):
    grid = (n // _RBLK,)
    return pl.pallas_call(
        _dist_topk_body,
        grid=grid,
        in_specs=[
            pl.BlockSpec((1, 1), lambda i: (0, 0)),
            pl.BlockSpec((n, xe.shape[1]), lambda i: (0, 0)),
            pl.BlockSpec((n, 1), lambda i: (0, 0)),
            pl.BlockSpec((1, n), lambda i: (0, 0)),
            pl.BlockSpec((_RBLK, n), lambda i: (i, 0)),
        ],
        out_specs=[
            pl.BlockSpec((_RBLK, _K), lambda i: (i, 0)),
            pl.BlockSpec((_RBLK, _K), lambda i: (i, 0)),
        ],
        out_shape=(jax.ShapeDtypeStruct((n, _K), jnp.float32),
                   jax.ShapeDtypeStruct((n, _K), jnp.int32)),
    )(tneg, xe, sqc, sqr, g)


def kernel(x, edge_index, W, b, temperature):
    n, _ = x.shape
    dout = W.shape[1]
    ei = edge_index.astype(jnp.int32)
    src, dst = ei[0], ei[1]

    degp = _sc_degree(dst, n)
    hn0, hn1, dinv = _tc_hn(x, W, degp, n, dout)
    dh = dout // 2
    accp = _sc_scatter_rows(src, dst, hn0, hn1, n, dh)

    g = jnp.asarray(_gumbel_table(n))

    b2 = b.reshape(1, dout)
    xe, sqc = _tc_combine(accp, hn0, hn1, dinv, b2, n, dout)
    sqr = sqc.reshape(1, n)

    tneg = (-temperature).reshape(1, 1)
    topvals, topidx = _tc_dist_topk(tneg, xe, sqc, sqr, g, n)

    ar = jnp.arange(n, dtype=jnp.int32)
    rows = jnp.repeat(ar, _K)
    edges = jnp.stack([topidx.reshape(-1), rows])
    edge_index_hat = jnp.concatenate([edges, jnp.stack([ar, ar])], axis=1)
    return (xe, edge_index_hat, topvals)
